# initial kernel scaffold (unmeasured)
import jax
import jax.numpy as jnp
from jax import lax
from jax.experimental import pallas as pl
from jax.experimental.pallas import tpu as pltpu

N_DEV = 8
SQ = 1024
SKV = 1024
H_PER = 8
DH = 128
D_MODEL = 1024
D_LOCAL = H_PER * DH
CHUNK = SQ // N_DEV
SCALE = 0.08838834764831843
NEG = -1e9


def kernel(x, Wq, K_ext, V_ext, Wo):
    x2 = x.reshape(SQ, D_MODEL)
    k2 = K_ext.reshape(SKV, D_LOCAL)
    v2 = V_ext.reshape(SKV, D_LOCAL)

    def body(x_ref, wq_hbm, k_ref, v_ref, wo_hbm, out_ref,
             wq_ref, wo_ref, q_ref, ctx_ref, bias_ref, send_ref, recv_ref,
             load_sems, rs_send_sems, rs_recv_sems, ag_send_sems,
             ag_recv_sems):
        idx = lax.axis_index("i")
        right = (idx + 1) % N_DEV
        left = (idx + N_DEV - 1) % N_DEV

        wq_dma = pltpu.make_async_copy(
            wq_hbm.at[:, pl.ds(idx * D_LOCAL, D_LOCAL)], wq_ref,
            load_sems.at[0])
        wq_dma.start()
        wo_dma = pltpu.make_async_copy(
            wo_hbm.at[pl.ds(idx * D_LOCAL, D_LOCAL), :], wo_ref,
            load_sems.at[1])
        wo_dma.start()

        barrier = pltpu.get_barrier_semaphore()
        for nbr in (left, right):
            pl.semaphore_signal(barrier, inc=1, device_id=(nbr,),
                                device_id_type=pl.DeviceIdType.MESH)
        pl.semaphore_wait(barrier, 2)

        qb = lax.broadcasted_iota(jnp.int32, (SQ, SKV), 0) // 64
        kb = lax.broadcasted_iota(jnp.int32, (SQ, SKV), 1) // 64
        mask = (qb == kb) | (kb == 0) | ((qb + kb) % 3 == 0)
        bias_ref[...] = jnp.where(mask, 0.0, NEG).astype(jnp.float32)

        wq_dma.wait()
        q_ref[...] = jnp.dot(
            x_ref[...].astype(jnp.bfloat16),
            wq_ref[...].astype(jnp.bfloat16),
            preferred_element_type=jnp.float32,
        ).astype(jnp.bfloat16)

        for h in range(H_PER):
            sl = pl.ds(h * DH, DH)
            qh = q_ref[:, sl]
            kh = k_ref[:, sl].astype(jnp.bfloat16)
            s = lax.dot_general(qh, kh, (((1,), (1,)), ((), ())),
                                preferred_element_type=jnp.float32)
            s = s * SCALE + bias_ref[...]
            m = jnp.max(s, axis=1, keepdims=True)
            w = jnp.exp(s - m)
            w = w / jnp.sum(w, axis=1, keepdims=True)
            vh = v_ref[:, sl].astype(jnp.bfloat16)
            ctx_ref[:, sl] = jnp.dot(
                w.astype(jnp.bfloat16), vh,
                preferred_element_type=jnp.float32,
            ).astype(jnp.bfloat16)

        wo_dma.wait()
        out_ref[...] = jnp.dot(ctx_ref[...], wo_ref[...].astype(jnp.bfloat16),
                               preferred_element_type=jnp.float32)

        for s_ in range(N_DEV - 1):
            chunk = (idx + N_DEV - s_) % N_DEV
            rows = pl.ds(chunk * CHUNK, CHUNK)
            if s_ == 0:
                send_ref[s_] = out_ref[rows, :]
            else:
                send_ref[s_] = recv_ref[s_ - 1] + out_ref[rows, :]
            rdma = pltpu.make_async_remote_copy(
                src_ref=send_ref.at[s_], dst_ref=recv_ref.at[s_],
                send_sem=rs_send_sems.at[s_], recv_sem=rs_recv_sems.at[s_],
                device_id=(right,), device_id_type=pl.DeviceIdType.MESH)
            rdma.start()
            rdma.wait()

        fc = (idx + 1) % N_DEV
        frows = pl.ds(fc * CHUNK, CHUNK)
        out_ref[frows, :] = out_ref[frows, :] + recv_ref[N_DEV - 2]

        for t in range(N_DEV - 1):
            pchunk = (idx + 1 + N_DEV - t) % N_DEV
            prows = pl.ds(pchunk * CHUNK, CHUNK)
            rdma = pltpu.make_async_remote_copy(
                src_ref=out_ref.at[prows, :], dst_ref=out_ref.at[prows, :],
                send_sem=ag_send_sems.at[t], recv_sem=ag_recv_sems.at[t],
                device_id=(right,), device_id_type=pl.DeviceIdType.MESH)
            rdma.start()
            rdma.wait()

    out = pl.pallas_call(
        body,
        out_shape=jax.ShapeDtypeStruct((SQ, D_MODEL), jnp.float32),
        in_specs=[
            pl.BlockSpec(memory_space=pltpu.VMEM),
            pl.BlockSpec(memory_space=pltpu.ANY),
            pl.BlockSpec(memory_space=pltpu.VMEM),
            pl.BlockSpec(memory_space=pltpu.VMEM),
            pl.BlockSpec(memory_space=pltpu.ANY),
        ],
        out_specs=pl.BlockSpec(memory_space=pltpu.VMEM),
        scratch_shapes=[
            pltpu.VMEM((D_MODEL, D_LOCAL), jnp.float32),
            pltpu.VMEM((D_LOCAL, D_MODEL), jnp.float32),
            pltpu.VMEM((SQ, D_LOCAL), jnp.bfloat16),
            pltpu.VMEM((SQ, D_LOCAL), jnp.bfloat16),
            pltpu.VMEM((SQ, SKV), jnp.float32),
            pltpu.VMEM((N_DEV - 1, CHUNK, D_MODEL), jnp.float32),
            pltpu.VMEM((N_DEV - 1, CHUNK, D_MODEL), jnp.float32),
            pltpu.SemaphoreType.DMA((2,)),
            pltpu.SemaphoreType.DMA((N_DEV - 1,)),
            pltpu.SemaphoreType.DMA((N_DEV - 1,)),
            pltpu.SemaphoreType.DMA((N_DEV - 1,)),
            pltpu.SemaphoreType.DMA((N_DEV - 1,)),
        ],
        compiler_params=pltpu.CompilerParams(collective_id=0),
    )(x2, Wq, k2, v2, Wo)
    return out.reshape(1, SQ, D_MODEL)


# baseline (device time: 139441 ns/iter reference)
import jax
import jax.numpy as jnp
from jax import lax
from jax.experimental import pallas as pl
from jax.experimental.pallas import tpu as pltpu

N_DEV = 8
SQ = 1024
SKV = 1024
H_PER = 8
DH = 128
D_MODEL = 1024
D_LOCAL = H_PER * DH
CHUNK = SQ // N_DEV
SCALE = 0.08838834764831843
NEG = -1e9


def kernel(x, Wq, K_ext, V_ext, Wo):
    x2 = x.reshape(SQ, D_MODEL)
    k2 = K_ext.reshape(SKV, D_LOCAL)
    v2 = V_ext.reshape(SKV, D_LOCAL)

    def body(x_ref, wq_hbm, k_ref, v_ref, wo_hbm, out_ref,
             wq_ref, wo_ref, q_ref, ctx_ref, bias_ref, send_ref, recv_ref,
             load_sems, rs_send_sems, rs_recv_sems, ag_send_sems,
             ag_recv_sems):
        idx = lax.axis_index("i")
        right = (idx + 1) % N_DEV
        left = (idx + N_DEV - 1) % N_DEV

        wq_dma = pltpu.make_async_copy(
            wq_hbm.at[:, pl.ds(idx * D_LOCAL, D_LOCAL)], wq_ref,
            load_sems.at[0])
        wq_dma.start()
        wo_dma = pltpu.make_async_copy(
            wo_hbm.at[pl.ds(idx * D_LOCAL, D_LOCAL), :], wo_ref,
            load_sems.at[1])
        wo_dma.start()

        barrier = pltpu.get_barrier_semaphore()
        for nbr in (left, right):
            pl.semaphore_signal(barrier, inc=1, device_id=(nbr,),
                                device_id_type=pl.DeviceIdType.MESH)
        pl.semaphore_wait(barrier, 2)

        qb = lax.broadcasted_iota(jnp.int32, (SQ, SKV), 0) // 64
        kb = lax.broadcasted_iota(jnp.int32, (SQ, SKV), 1) // 64
        mask = (qb == kb) | (kb == 0) | ((qb + kb) % 3 == 0)
        bias_ref[...] = jnp.where(mask, 0.0, NEG).astype(jnp.float32)

        wq_dma.wait()
        q_ref[...] = jnp.dot(
            x_ref[...].astype(jnp.bfloat16),
            wq_ref[...].astype(jnp.bfloat16),
            preferred_element_type=jnp.float32,
        ).astype(jnp.bfloat16)

        for h in range(H_PER):
            sl = pl.ds(h * DH, DH)
            qh = q_ref[:, sl]
            kh = k_ref[:, sl].astype(jnp.bfloat16)
            s = lax.dot_general(qh, kh, (((1,), (1,)), ((), ())),
                                preferred_element_type=jnp.float32)
            s = s * SCALE + bias_ref[...]
            m = jnp.max(s, axis=1, keepdims=True)
            w = jnp.exp(s - m)
            w = w / jnp.sum(w, axis=1, keepdims=True)
            vh = v_ref[:, sl].astype(jnp.bfloat16)
            ctx_ref[:, sl] = jnp.dot(
                w.astype(jnp.bfloat16), vh,
                preferred_element_type=jnp.float32,
            ).astype(jnp.bfloat16)

        wo_dma.wait()
        out_ref[...] = jnp.dot(ctx_ref[...], wo_ref[...].astype(jnp.bfloat16),
                               preferred_element_type=jnp.float32)

        for s_ in range(N_DEV - 1):
            chunk = (idx + N_DEV - s_) % N_DEV
            rows = pl.ds(chunk * CHUNK, CHUNK)
            if s_ == 0:
                send_ref[s_] = out_ref[rows, :]
            else:
                send_ref[s_] = recv_ref[s_ - 1] + out_ref[rows, :]
            rdma = pltpu.make_async_remote_copy(
                src_ref=send_ref.at[s_], dst_ref=recv_ref.at[s_],
                send_sem=rs_send_sems.at[s_], recv_sem=rs_recv_sems.at[s_],
                device_id=(right,), device_id_type=pl.DeviceIdType.MESH)
            rdma.start()
            rdma.wait()

        fc = (idx + 1) % N_DEV
        frows = pl.ds(fc * CHUNK, CHUNK)
        out_ref[frows, :] = out_ref[frows, :] + recv_ref[N_DEV - 2]

        for t in range(N_DEV - 1):
            pchunk = (idx + 1 + N_DEV - t) % N_DEV
            prows = pl.ds(pchunk * CHUNK, CHUNK)
            rdma = pltpu.make_async_remote_copy(
                src_ref=out_ref.at[prows, :], dst_ref=out_ref.at[prows, :],
                send_sem=ag_send_sems.at[t], recv_sem=ag_recv_sems.at[t],
                device_id=(right,), device_id_type=pl.DeviceIdType.MESH)
            rdma.start()
            rdma.wait()

    out = pl.pallas_call(
        body,
        out_shape=jax.ShapeDtypeStruct((SQ, D_MODEL), jnp.float32),
        in_specs=[
            pl.BlockSpec(memory_space=pltpu.VMEM),
            pl.BlockSpec(memory_space=pl.ANY),
            pl.BlockSpec(memory_space=pltpu.VMEM),
            pl.BlockSpec(memory_space=pltpu.VMEM),
            pl.BlockSpec(memory_space=pl.ANY),
        ],
        out_specs=pl.BlockSpec(memory_space=pltpu.VMEM),
        scratch_shapes=[
            pltpu.VMEM((D_MODEL, D_LOCAL), jnp.float32),
            pltpu.VMEM((D_LOCAL, D_MODEL), jnp.float32),
            pltpu.VMEM((SQ, D_LOCAL), jnp.bfloat16),
            pltpu.VMEM((SQ, D_LOCAL), jnp.bfloat16),
            pltpu.VMEM((SQ, SKV), jnp.float32),
            pltpu.VMEM((N_DEV - 1, CHUNK, D_MODEL), jnp.float32),
            pltpu.VMEM((N_DEV - 1, CHUNK, D_MODEL), jnp.float32),
            pltpu.SemaphoreType.DMA((2,)),
            pltpu.SemaphoreType.DMA((N_DEV - 1,)),
            pltpu.SemaphoreType.DMA((N_DEV - 1,)),
            pltpu.SemaphoreType.DMA((N_DEV - 1,)),
            pltpu.SemaphoreType.DMA((N_DEV - 1,)),
        ],
        compiler_params=pltpu.CompilerParams(collective_id=0),
    )(x2, Wq, k2, v2, Wo)
    return out.reshape(1, SQ, D_MODEL)


# device time: 68724 ns/iter; 2.0290x vs baseline; 2.0290x over previous
import jax
import jax.numpy as jnp
from jax import lax
from jax.experimental import pallas as pl
from jax.experimental.pallas import tpu as pltpu

N_DEV = 8
SQ = 1024
SKV = 1024
H_PER = 8
DH = 128
D_MODEL = 1024
D_LOCAL = H_PER * DH
CHUNK = SQ // N_DEV
SCALE = 0.08838834764831843
NEG = -1e9


def kernel(x, Wq, K_ext, V_ext, Wo):
    x2 = x.reshape(SQ, D_MODEL)
    k2 = K_ext.reshape(SKV, D_LOCAL)
    v2 = V_ext.reshape(SKV, D_LOCAL)

    def body(x_ref, wq_hbm, k_ref, v_ref, wo_hbm, out_ref,
             wq_ref, wo_ref, q_ref, ctx_ref, bias_ref,
             partial_ref, rs_recv, ag_buf,
             load_sems, rs_send_sems, rs_recv_sems, ag_send_sems,
             ag_recv_sems):
        idx = lax.axis_index("i")

        wq_dma = pltpu.make_async_copy(
            wq_hbm.at[:, pl.ds(idx * D_LOCAL, D_LOCAL)], wq_ref,
            load_sems.at[0])
        wq_dma.start()
        wo_dma = pltpu.make_async_copy(
            wo_hbm.at[pl.ds(idx * D_LOCAL, D_LOCAL), :], wo_ref,
            load_sems.at[1])
        wo_dma.start()

        barrier = pltpu.get_barrier_semaphore()
        for j in range(1, N_DEV):
            pl.semaphore_signal(barrier, inc=1,
                                device_id=((idx + j) % N_DEV,),
                                device_id_type=pl.DeviceIdType.MESH)
        pl.semaphore_wait(barrier, N_DEV - 1)

        qb = lax.broadcasted_iota(jnp.int32, (SQ, SKV), 0) // 64
        kb = lax.broadcasted_iota(jnp.int32, (SQ, SKV), 1) // 64
        mask = (qb == kb) | (kb == 0) | ((qb + kb) % 3 == 0)
        bias_ref[...] = jnp.where(mask, 0.0, NEG).astype(jnp.float32)

        wq_dma.wait()
        q_ref[...] = jnp.dot(
            x_ref[...].astype(jnp.bfloat16),
            wq_ref[...].astype(jnp.bfloat16),
            preferred_element_type=jnp.float32,
        ).astype(jnp.bfloat16)

        for h in range(H_PER):
            sl = pl.ds(h * DH, DH)
            qh = q_ref[:, sl]
            kh = k_ref[:, sl].astype(jnp.bfloat16)
            s = lax.dot_general(qh, kh, (((1,), (1,)), ((), ())),
                                preferred_element_type=jnp.float32)
            s = s * SCALE + bias_ref[...]
            m = jnp.max(s, axis=1, keepdims=True)
            w = jnp.exp(s - m)
            w = w / jnp.sum(w, axis=1, keepdims=True)
            vh = v_ref[:, sl].astype(jnp.bfloat16)
            ctx_ref[:, sl] = jnp.dot(
                w.astype(jnp.bfloat16), vh,
                preferred_element_type=jnp.float32,
            ).astype(jnp.bfloat16)

        wo_dma.wait()
        partial_ref[...] = jnp.dot(
            ctx_ref[...], wo_ref[...].astype(jnp.bfloat16),
            preferred_element_type=jnp.float32,
        ).astype(jnp.bfloat16).reshape(N_DEV, CHUNK, D_MODEL)

        rs = []
        for j in range(N_DEV - 1):
            p = (idx + 1 + j) % N_DEV
            r = pltpu.make_async_remote_copy(
                src_ref=partial_ref.at[p], dst_ref=rs_recv.at[j],
                send_sem=rs_send_sems.at[j], recv_sem=rs_recv_sems.at[j],
                device_id=(p,), device_id_type=pl.DeviceIdType.MESH)
            r.start()
            rs.append(r)
        for r in rs:
            r.wait_recv()

        red = (jnp.sum(rs_recv[...].astype(jnp.float32), axis=0)
               + partial_ref[idx].astype(jnp.float32))
        ag_buf[idx] = red.astype(jnp.bfloat16)

        ag = []
        for j in range(N_DEV - 1):
            p = (idx + 1 + j) % N_DEV
            a = pltpu.make_async_remote_copy(
                src_ref=ag_buf.at[idx], dst_ref=ag_buf.at[idx],
                send_sem=ag_send_sems.at[j], recv_sem=ag_recv_sems.at[j],
                device_id=(p,), device_id_type=pl.DeviceIdType.MESH)
            a.start()
            ag.append(a)
        for a in ag:
            a.wait_recv()

        out_ref[...] = ag_buf[...].astype(jnp.float32).reshape(SQ, D_MODEL)

        for r in rs:
            r.wait_send()
        for a in ag:
            a.wait_send()

    out = pl.pallas_call(
        body,
        out_shape=jax.ShapeDtypeStruct((SQ, D_MODEL), jnp.float32),
        in_specs=[
            pl.BlockSpec(memory_space=pltpu.VMEM),
            pl.BlockSpec(memory_space=pl.ANY),
            pl.BlockSpec(memory_space=pltpu.VMEM),
            pl.BlockSpec(memory_space=pltpu.VMEM),
            pl.BlockSpec(memory_space=pl.ANY),
        ],
        out_specs=pl.BlockSpec(memory_space=pltpu.VMEM),
        scratch_shapes=[
            pltpu.VMEM((D_MODEL, D_LOCAL), jnp.float32),
            pltpu.VMEM((D_LOCAL, D_MODEL), jnp.float32),
            pltpu.VMEM((SQ, D_LOCAL), jnp.bfloat16),
            pltpu.VMEM((SQ, D_LOCAL), jnp.bfloat16),
            pltpu.VMEM((SQ, SKV), jnp.float32),
            pltpu.VMEM((N_DEV, CHUNK, D_MODEL), jnp.bfloat16),
            pltpu.VMEM((N_DEV - 1, CHUNK, D_MODEL), jnp.bfloat16),
            pltpu.VMEM((N_DEV, CHUNK, D_MODEL), jnp.bfloat16),
            pltpu.SemaphoreType.DMA((2,)),
            pltpu.SemaphoreType.DMA((N_DEV - 1,)),
            pltpu.SemaphoreType.DMA((N_DEV - 1,)),
            pltpu.SemaphoreType.DMA((N_DEV - 1,)),
            pltpu.SemaphoreType.DMA((N_DEV - 1,)),
        ],
        compiler_params=pltpu.CompilerParams(collective_id=0),
    )(x2, Wq, k2, v2, Wo)
    return out.reshape(1, SQ, D_MODEL)
